# baseline (speedup 1.0000x reference)
import functools

import jax
import jax.numpy as jnp
from jax import lax
from jax.experimental import pallas as pl
from jax.experimental.pallas import tpu as pltpu
from jax.experimental.pallas import tpu_sc as plsc

VOCAB = 100000
DIM = 128
L_CTX = 200
L_TGT = 20

NTILE = 16
ROWS = DIM // NTILE
CPAD = 208
TTOT = ROWS * L_TGT


def _vi(x):
    return lax.broadcast_in_dim(jnp.asarray(x, jnp.int32), (16,), ())


def _vf(x):
    return lax.broadcast_in_dim(jnp.asarray(x, jnp.float32), (16,), ())


def _body(ctx_hbm, tgt_hbm, words_hbm, tree_hbm, out_hbm,
          cidx, coff, cval, tidx, toff, tval, outacc, shared, red, outbuf,
          mred, sem):
    sid = lax.axis_index("s")
    d0 = sid * ROWS
    lane = lax.iota(jnp.int32, 16)
    m8 = lane < _vi(8)
    zi = _vi(0)
    zf = _vf(0.0)

    cps = []
    for r in range(ROWS):
        cps.append(pltpu.async_copy(
            ctx_hbm.at[pl.ds((d0 + r) * L_CTX, L_CTX)],
            cidx.at[pl.ds(r * CPAD, L_CTX)], sem))
    cps.append(pltpu.async_copy(
        tgt_hbm.at[pl.ds(d0 * L_TGT, TTOT)], tidx, sem))
    for cp in cps:
        cp.wait()

    for r in range(ROWS):
        base = _vi((d0 + r) * VOCAB)
        for i in range(13):
            v = cidx[pl.ds(r * CPAD + i * 16, 16)]
            off = v + base
            if i == 12:
                off = jnp.where(m8, off, zi)
            coff[pl.ds(r * CPAD + i * 16, 16)] = off

    vocab_v = _vi(VOCAB)
    d0v = _vi(d0)
    for i in range(10):
        lo = (i * 16) // L_TGT
        hi = (i * 16 + 15) // L_TGT
        if hi == lo:
            dl_c = _vi(lo)
        else:
            bnd = (lo + 1) * L_TGT - i * 16
            dl_c = jnp.where(lane >= _vi(bnd), _vi(hi), _vi(lo))
        off = tidx[pl.ds(i * 16, 16)] + (d0v + dl_c) * vocab_v
        toff[pl.ds(i * 16, 16)] = off

    gs = []
    for r in range(ROWS):
        b = r * CPAD
        gs.append(pltpu.async_copy(
            words_hbm.at[coff.at[pl.ds(b, 128)]],
            cval.at[pl.ds(b, 128)], sem))
        gs.append(pltpu.async_copy(
            words_hbm.at[coff.at[pl.ds(b + 128, 80)]],
            cval.at[pl.ds(b + 128, 80)], sem))
    gs.append(pltpu.async_copy(
        tree_hbm.at[toff.at[pl.ds(0, 128)]], tval.at[pl.ds(0, 128)], sem))
    gs.append(pltpu.async_copy(
        tree_hbm.at[toff.at[pl.ds(128, 32)]], tval.at[pl.ds(128, 32)], sem))
    for cp in gs:
        cp.wait()

    scale_v = _vf(1.0 / L_CTX)
    means = []
    for r in range(ROWS):
        acc = zf
        for i in range(13):
            v = cval[pl.ds(r * CPAD + i * 16, 16)]
            if i == 12:
                v = jnp.where(m8, v, zf)
            acc = acc + v
        for k in (8, 4, 2, 1):
            mred[pl.ds(0, 16)] = acc
            acc = acc + plsc.load_gather(mred, [lane ^ _vi(k)])
        means.append(acc * scale_v)

    outacc[pl.ds(0, 16)] = zf
    outacc[pl.ds(16, 16)] = zf
    for i in range(10):
        lo = (i * 16) // L_TGT
        hi = (i * 16 + 15) // L_TGT
        pos = lane + _vi(i * 16)
        if hi == lo:
            dl_c = _vi(lo)
            mv = means[lo]
        else:
            bnd = (lo + 1) * L_TGT - i * 16
            mhi = lane >= _vi(bnd)
            dl_c = jnp.where(mhi, _vi(hi), _vi(lo))
            mv = jnp.where(mhi, means[hi], means[lo])
        t_c = pos - dl_c * _vi(L_TGT)
        prod = mv * tval[pl.ds(i * 16, 16)]
        plsc.addupdate_scatter(outacc, [t_c], prod)

    pltpu.sync_copy(outacc, shared.at[pl.ds(32 * sid, 32)])
    plsc.subcore_barrier()

    pltpu.sync_copy(shared, red)
    a0 = _vf(0.0)
    a1 = _vf(0.0)
    for s in range(NTILE):
        a0 = a0 + red[pl.ds(32 * s, 16)]
        a1 = a1 + red[pl.ds(32 * s + 16, 16)]
    onev = _vf(1.0)
    outbuf[pl.ds(0, 16)] = onev / (onev + jnp.exp(-a0))
    outbuf[pl.ds(16, 16)] = onev / (onev + jnp.exp(-a1))

    @pl.when(sid == 0)
    def _():
        pltpu.sync_copy(outbuf, out_hbm)


_mesh = plsc.VectorSubcoreMesh(
    core_axis_name="c", subcore_axis_name="s", num_cores=1)

_sc_call = functools.partial(
    pl.kernel,
    out_type=jax.ShapeDtypeStruct((32,), jnp.float32),
    mesh=_mesh,
    compiler_params=pltpu.CompilerParams(needs_layout_passes=False),
    scratch_types=[
        pltpu.VMEM((ROWS * CPAD,), jnp.int32),
        pltpu.VMEM((ROWS * CPAD,), jnp.int32),
        pltpu.VMEM((ROWS * CPAD,), jnp.float32),
        pltpu.VMEM((TTOT,), jnp.int32),
        pltpu.VMEM((TTOT,), jnp.int32),
        pltpu.VMEM((TTOT,), jnp.float32),
        pltpu.VMEM((32,), jnp.float32),
        pltpu.VMEM_SHARED((NTILE * 32,), jnp.float32),
        pltpu.VMEM((NTILE * 32,), jnp.float32),
        pltpu.VMEM((32,), jnp.float32),
        pltpu.VMEM((16,), jnp.float32),
        pltpu.SemaphoreType.DMA,
    ],
)(_body)


@jax.jit
def kernel(context_gather_content, target_tree_gather_content, words, tree):
    out = _sc_call(
        context_gather_content.reshape(-1),
        target_tree_gather_content.reshape(-1),
        words.reshape(-1),
        tree.reshape(-1),
    )
    return out[:L_TGT]
